# Initial kernel scaffold; baseline (speedup 1.0000x reference)
import jax
import jax.numpy as jnp
from jax.experimental import pallas as pl


def kernel(prediction, target, no_bg):
    raise NotImplementedError("write your pallas kernel here")


# speedup vs baseline: 6.9443x; 6.9443x over previous
import functools

import jax
import jax.numpy as jnp
from jax import lax
from jax.experimental import pallas as pl
from jax.experimental.pallas import tpu as pltpu

_LAMBDA = 300.0
_NSEG = 64


def _huber(x):
    ax = jnp.abs(x)
    return jnp.where(ax < 1.0, 0.5 * x * x, ax - 0.5)


def _loss_kernel(nobg_ref, pred_ref, tgt_ref, out_ref, stats_ref, sep_ref,
                 *, nblk, hw, nbatch):
    p = pl.program_id(0)
    i = pl.program_id(1)

    pred_all = pred_ref[...]
    tgt_all = tgt_ref[...]

    @pl.when(p == 0)
    def _phase0():
        @pl.when(i == 0)
        def _init():
            stats_ref[...] = jnp.zeros_like(stats_ref)

        for bb in range(nbatch):
            predb = pred_all[3 * bb:3 * bb + 3, :]
            tgtb = tgt_all[3 * bb:3 * bb + 3, :]
            idv = tgtb[0:1, :] * 16 + tgtb[1:2, :] * 4 + tgtb[2:3, :]
            blk = predb.shape[1]
            oh = (lax.broadcasted_iota(jnp.int32, (_NSEG, blk), 0)
                  == idv).astype(jnp.float32)
            cnt = jnp.sum(oh, axis=1, keepdims=True)
            spred = lax.dot_general(
                oh, predb, (((1,), (1,)), ((), ())),
                preferred_element_type=jnp.float32)
            h0 = jnp.sum(_huber(predb), axis=0, keepdims=True)
            h255 = jnp.sum(_huber(predb - 255.0), axis=0, keepdims=True)
            hsel = jnp.where(idv == 0, h0, h255)
            hub = lax.dot_general(
                oh, hsel, (((1,), (1,)), ((), ())),
                preferred_element_type=jnp.float32)
            stats_ref[bb, :, 0:1] += cnt
            stats_ref[bb, :, 1:4] += spred
            stats_ref[bb, :, 4:5] += hub

    @pl.when(p == 1)
    def _phase1():
        @pl.when(i == 0)
        def _init():
            sep_ref[...] = jnp.zeros_like(sep_ref)

        for bb in range(nbatch):
            predb = pred_all[3 * bb:3 * bb + 3, :]
            tgtb = tgt_all[3 * bb:3 * bb + 3, :]
            idv = tgtb[0:1, :] * 16 + tgtb[1:2, :] * 4 + tgtb[2:3, :]
            cnt = stats_ref[bb, :, 0:1]
            spred = stats_ref[bb, :, 1:4]
            size_safe = jnp.maximum(cnt, 1.0)
            means = spred / size_safe
            mnorm = jnp.sum(means * means, axis=1, keepdims=True)
            G = lax.dot_general(
                means, predb, (((1,), (0,)), ((), ())),
                preferred_element_type=jnp.float32)
            pnorm = jnp.sum(predb * predb, axis=0, keepdims=True)
            dist = mnorm + pnorm - 2.0 * G
            T = _LAMBDA / (1.0 + dist)
            mask0 = (idv == 0).astype(jnp.float32)
            S = lax.dot_general(
                T, mask0, (((1,), (1,)), ((), ())),
                preferred_element_type=jnp.float32)
            rs = jnp.sum(T, axis=1, keepdims=True)
            sep_ref[bb, :, 0:1] += S
            sep_ref[bb, :, 1:2] += rs

    @pl.when(jnp.logical_and(p == 1, i == nblk - 1))
    def _finalize():
        total = jnp.zeros((1, 1), dtype=jnp.float32)
        rowidx = lax.broadcasted_iota(jnp.int32, (_NSEG, 1), 0)
        for bb in range(nbatch):
            cnt = stats_ref[bb, :, 0:1]
            hub = stats_ref[bb, :, 4:5]
            S = sep_ref[bb, :, 0:1]
            rs = sep_ref[bb, :, 1:2]
            nobg = nobg_ref[bb, 0]
            present = cnt > 0.0
            size_safe = jnp.maximum(cnt, 1.0)
            var_loss = hub / (size_safe * 3.0)
            w = 10.0 * lax.rsqrt(size_safe)
            cnt0 = cnt[0:1, :]
            bg_present = cnt0 > 0.0
            use0 = jnp.logical_and(bg_present, nobg == 0)
            n_non = float(hw) - cnt0
            sep0 = (rs[0:1, :] - S[0:1, :]) / jnp.maximum(n_non, 1.0)
            contrib0 = (jnp.where(use0, var_loss[0:1, :], 0.0)
                        + jnp.where(jnp.logical_and(use0, n_non > 0.0),
                                    w[0:1, :] * sep0, 0.0))
            sepj = S / jnp.maximum(cnt0, 1.0)
            contribj = (jnp.where(present, var_loss, 0.0)
                        + jnp.where(jnp.logical_and(present, bg_present),
                                    w * sepj, 0.0))
            contrib = jnp.where(rowidx == 0, 0.0, contribj)
            loss_b = jnp.sum(contrib) + jnp.sum(contrib0)
            ctv = jnp.where(rowidx == 0,
                            jnp.broadcast_to(use0.astype(jnp.float32), (_NSEG, 1)),
                            present.astype(jnp.float32))
            ct = jnp.maximum(jnp.sum(ctv), 1.0)
            total += loss_b / ct
        out_ref[...] = total / float(nbatch)


def kernel(prediction, target, no_bg):
    prediction = prediction.astype(jnp.float32)
    B, C, H, W = prediction.shape
    HW = H * W
    BLK = 4096
    nblk = HW // BLK
    predr = prediction.reshape(B * C, HW)
    tgtr = target.astype(jnp.int32).reshape(B * C, HW)
    nobg = no_bg.astype(jnp.int32).reshape(B, 1)

    out = pl.pallas_call(
        functools.partial(_loss_kernel, nblk=nblk, hw=HW, nbatch=B),
        grid=(2, nblk),
        in_specs=[
            pl.BlockSpec(memory_space=pltpu.SMEM),
            pl.BlockSpec((B * C, BLK), lambda p, i: (0, i)),
            pl.BlockSpec((B * C, BLK), lambda p, i: (0, i)),
        ],
        out_specs=pl.BlockSpec((1, 1), lambda p, i: (0, 0)),
        out_shape=jax.ShapeDtypeStruct((1, 1), jnp.float32),
        scratch_shapes=[
            pltpu.VMEM((B, _NSEG, 8), jnp.float32),
            pltpu.VMEM((B, _NSEG, 8), jnp.float32),
        ],
    )(nobg, predr, tgtr)
    return out[0, 0]
